# baseline (speedup 1.0000x reference)
import jax
import jax.numpy as jnp
from jax.experimental import pallas as pl
from jax.experimental.pallas import tpu as pltpu

_NUM_CLASSES_OUT = 64


def _tile_schedule(n):
    for c in (1024, 512, 256, 128, 64, 32, 16, 8, 1):
        if n % c == 0:
            return [c] * (n // c)


def _fused_body(w1_ref, b1_ref, wl_ref, bl_ref,
                hw1_ref, hb1_ref, hw2_ref, hb2_ref, hw3_ref, hb3_ref,
                scx_ref, scadj_hbm, fcx_ref, fcadj_hbm, pool_ref,
                o_ref, abuf_a, abuf_b, a16, sem, h16, xr, pooled,
                x16a, x16b, pool16):
    n = abuf_a.shape[0]
    f_pad = w1_ref.shape[1] // 2
    h_pad = w1_ref.shape[2]
    num_extra = wl_ref.shape[1]

    tiles = _tile_schedule(n)
    offs = [sum(tiles[:i]) for i in range(len(tiles))]
    nt = len(tiles)

    def copy(adj_hbm, abuf, bidx, t):
        return pltpu.make_async_copy(
            adj_hbm.at[pl.ds(offs[t], tiles[t])],
            abuf.at[pl.ds(offs[t], tiles[t])],
            sem.at[bidx, t])

    for t in range(nt):
        copy(scadj_hbm, abuf_a, 0, t).start()

    pool16[...] = pool_ref[...].astype(jnp.bfloat16)
    x16a[...] = scx_ref[...].astype(jnp.bfloat16)
    x16b[...] = fcx_ref[...].astype(jnp.bfloat16)

    xr[:, 0:h_pad] = jnp.dot(x16a[...], w1_ref[0, f_pad:],
                             preferred_element_type=jnp.float32) + b1_ref[0]
    xr[:, h_pad:2 * h_pad] = jnp.dot(x16b[...], w1_ref[1, f_pad:],
                                     preferred_element_type=jnp.float32) + b1_ref[1]

    def run_branch(adj_hbm, abuf, x16, bidx):
        w1l = w1_ref[bidx, :f_pad]

        for t in range(nt):
            copy(adj_hbm, abuf, bidx, t).wait()
            rows = pl.ds(offs[t], tiles[t])
            a16[rows, :] = abuf[rows, :].astype(jnp.bfloat16)
            agg = jnp.dot(a16[rows, :], x16[...],
                          preferred_element_type=jnp.float32)
            z = (jnp.dot(agg.astype(jnp.bfloat16), w1l,
                         preferred_element_type=jnp.float32)
                 + xr[rows, bidx * h_pad:(bidx + 1) * h_pad])
            h16[rows, :] = jnp.maximum(z, 0.0).astype(jnp.bfloat16)
            if bidx == 0:
                copy(fcadj_hbm, abuf_b, 1, t).start()

        for layer in range(num_extra):
            wlb = wl_ref[bidx, layer]
            blb = bl_ref[bidx, layer]
            agg = jnp.dot(a16[...], h16[...],
                          preferred_element_type=jnp.float32)
            z = (jnp.dot(agg.astype(jnp.bfloat16), wlb[:h_pad],
                         preferred_element_type=jnp.float32)
                 + jnp.dot(h16[...], wlb[h_pad:],
                           preferred_element_type=jnp.float32)
                 + blb)
            h16[...] = jnp.maximum(z, 0.0).astype(jnp.bfloat16)

        pooled[:, bidx * h_pad:(bidx + 1) * h_pad] = jnp.dot(
            pool16[...], h16[...], preferred_element_type=jnp.float32)

    run_branch(scadj_hbm, abuf_a, x16a, 0)
    run_branch(fcadj_hbm, abuf_b, x16b, 1)

    t1 = jnp.maximum(jnp.dot(pooled[...], hw1_ref[...],
                             preferred_element_type=jnp.float32)
                     + hb1_ref[...], 0.0)
    t2 = jnp.maximum(jnp.dot(t1, hw2_ref[...],
                             preferred_element_type=jnp.float32)
                     + hb2_ref[...], 0.0)
    logits = jnp.dot(t2, hw3_ref[...],
                     preferred_element_type=jnp.float32) + hb3_ref[...]
    col = jax.lax.broadcasted_iota(jnp.int32, logits.shape, 1)
    logits = jnp.where(col < _NUM_CLASSES_OUT, logits, -1e30)
    m = jnp.max(logits, axis=-1, keepdims=True)
    z = logits - m
    lse = jnp.log(jnp.sum(jnp.exp(z), axis=-1, keepdims=True))
    o_ref[...] = (z - lse)[:, :_NUM_CLASSES_OUT]


def kernel(w1, b1, wl, bl, head_w1, head_b1, head_w2, head_b2,
           head_w3, head_b3, sc_x, sc_adj, fc_x, fc_adj, pool_mat):
    n = sc_x.shape[0]
    g = pool_mat.shape[0]
    h_pad = w1.shape[2]

    nt = len(_tile_schedule(n))

    vmem = pl.BlockSpec(memory_space=pltpu.MemorySpace.VMEM)
    hbm = pl.BlockSpec(memory_space=pl.ANY)

    out = pl.pallas_call(
        _fused_body,
        out_shape=jax.ShapeDtypeStruct((g, _NUM_CLASSES_OUT), jnp.float32),
        in_specs=[vmem, vmem, vmem, vmem,
                  vmem, vmem, vmem, vmem, vmem, vmem,
                  vmem, hbm, vmem, hbm, vmem],
        out_specs=vmem,
        scratch_shapes=[
            pltpu.VMEM((n, n), jnp.float32),
            pltpu.VMEM((n, n), jnp.float32),
            pltpu.VMEM((n, n), jnp.bfloat16),
            pltpu.SemaphoreType.DMA((2, nt)),
            pltpu.VMEM((n, h_pad), jnp.bfloat16),
            pltpu.VMEM((n, 2 * h_pad), jnp.float32),
            pltpu.VMEM((g, 2 * h_pad), jnp.float32),
            pltpu.VMEM((n, w1.shape[1] // 2), jnp.bfloat16),
            pltpu.VMEM((n, w1.shape[1] // 2), jnp.bfloat16),
            pltpu.VMEM((g, n), jnp.bfloat16),
        ],
        name="graphsage_fused",
    )(w1, b1, wl, bl, head_w1, head_b1, head_w2, head_b2, head_w3, head_b3,
      sc_x, sc_adj, fc_x, fc_adj, pool_mat)
    return out


# speedup vs baseline: 1.0750x; 1.0750x over previous
import jax
import jax.numpy as jnp
from jax.experimental import pallas as pl
from jax.experimental.pallas import tpu as pltpu

_NUM_CLASSES_OUT = 64


def _tile_schedule(n):
    for c in (512, 256, 128, 64, 32, 16, 8, 1):
        if n % c == 0:
            return [c] * (n // c)


def _fused_body(w1_ref, b1_ref, wl_ref, bl_ref,
                hw1_ref, hb1_ref, hw2_ref, hb2_ref, hw3_ref, hb3_ref,
                scx_ref, scadj_hbm, fcx_ref, fcadj_hbm, pool_ref,
                o_ref, abuf_a, abuf_b, a16, sem, h16, xr, pooled,
                x16a, x16b, pool16):
    n = abuf_a.shape[0]
    f_pad = w1_ref.shape[1] // 2
    h_pad = w1_ref.shape[2]
    num_extra = wl_ref.shape[1]

    tiles = _tile_schedule(n)
    offs = [sum(tiles[:i]) for i in range(len(tiles))]
    nt = len(tiles)

    def copy(adj_hbm, abuf, bidx, t):
        return pltpu.make_async_copy(
            adj_hbm.at[pl.ds(offs[t], tiles[t])],
            abuf.at[pl.ds(offs[t], tiles[t])],
            sem.at[bidx, t])

    for t in range(nt):
        copy(scadj_hbm, abuf_a, 0, t).start()

    pool16[...] = pool_ref[...].astype(jnp.bfloat16)
    x16a[...] = scx_ref[...].astype(jnp.bfloat16)
    x16b[...] = fcx_ref[...].astype(jnp.bfloat16)

    xr[:, 0:h_pad] = jnp.dot(x16a[...], w1_ref[0, f_pad:],
                             preferred_element_type=jnp.float32) + b1_ref[0]
    xr[:, h_pad:2 * h_pad] = jnp.dot(x16b[...], w1_ref[1, f_pad:],
                                     preferred_element_type=jnp.float32) + b1_ref[1]

    def run_branch(adj_hbm, abuf, x16, bidx):
        w1l = w1_ref[bidx, :f_pad]

        for t in range(nt):
            copy(adj_hbm, abuf, bidx, t).wait()
            rows = pl.ds(offs[t], tiles[t])
            a16[rows, :] = abuf[rows, :].astype(jnp.bfloat16)
            agg = jnp.dot(a16[rows, :], x16[...],
                          preferred_element_type=jnp.float32)
            z = (jnp.dot(agg.astype(jnp.bfloat16), w1l,
                         preferred_element_type=jnp.float32)
                 + xr[rows, bidx * h_pad:(bidx + 1) * h_pad])
            h16[rows, :] = jnp.maximum(z, 0.0).astype(jnp.bfloat16)
            if bidx == 0:
                copy(fcadj_hbm, abuf_b, 1, t).start()

        for layer in range(num_extra):
            wlb = wl_ref[bidx, layer]
            blb = bl_ref[bidx, layer]
            agg = jnp.dot(a16[...], h16[...],
                          preferred_element_type=jnp.float32)
            z = (jnp.dot(agg.astype(jnp.bfloat16), wlb[:h_pad],
                         preferred_element_type=jnp.float32)
                 + jnp.dot(h16[...], wlb[h_pad:],
                           preferred_element_type=jnp.float32)
                 + blb)
            h16[...] = jnp.maximum(z, 0.0).astype(jnp.bfloat16)

        pooled[:, bidx * h_pad:(bidx + 1) * h_pad] = jnp.dot(
            pool16[...], h16[...], preferred_element_type=jnp.float32)

    run_branch(scadj_hbm, abuf_a, x16a, 0)
    run_branch(fcadj_hbm, abuf_b, x16b, 1)

    t1 = jnp.maximum(jnp.dot(pooled[...], hw1_ref[...],
                             preferred_element_type=jnp.float32)
                     + hb1_ref[...], 0.0)
    t2 = jnp.maximum(jnp.dot(t1, hw2_ref[...],
                             preferred_element_type=jnp.float32)
                     + hb2_ref[...], 0.0)
    logits = jnp.dot(t2, hw3_ref[...],
                     preferred_element_type=jnp.float32) + hb3_ref[...]
    col = jax.lax.broadcasted_iota(jnp.int32, logits.shape, 1)
    logits = jnp.where(col < _NUM_CLASSES_OUT, logits, -1e30)
    m = jnp.max(logits, axis=-1, keepdims=True)
    z = logits - m
    lse = jnp.log(jnp.sum(jnp.exp(z), axis=-1, keepdims=True))
    o_ref[...] = (z - lse)[:, :_NUM_CLASSES_OUT]


def kernel(w1, b1, wl, bl, head_w1, head_b1, head_w2, head_b2,
           head_w3, head_b3, sc_x, sc_adj, fc_x, fc_adj, pool_mat):
    n = sc_x.shape[0]
    g = pool_mat.shape[0]
    h_pad = w1.shape[2]

    nt = len(_tile_schedule(n))

    vmem = pl.BlockSpec(memory_space=pltpu.MemorySpace.VMEM)
    hbm = pl.BlockSpec(memory_space=pl.ANY)

    out = pl.pallas_call(
        _fused_body,
        out_shape=jax.ShapeDtypeStruct((g, _NUM_CLASSES_OUT), jnp.float32),
        in_specs=[vmem, vmem, vmem, vmem,
                  vmem, vmem, vmem, vmem, vmem, vmem,
                  vmem, hbm, vmem, hbm, vmem],
        out_specs=vmem,
        scratch_shapes=[
            pltpu.VMEM((n, n), jnp.float32),
            pltpu.VMEM((n, n), jnp.float32),
            pltpu.VMEM((n, n), jnp.bfloat16),
            pltpu.SemaphoreType.DMA((2, nt)),
            pltpu.VMEM((n, h_pad), jnp.bfloat16),
            pltpu.VMEM((n, 2 * h_pad), jnp.float32),
            pltpu.VMEM((g, 2 * h_pad), jnp.float32),
            pltpu.VMEM((n, w1.shape[1] // 2), jnp.bfloat16),
            pltpu.VMEM((n, w1.shape[1] // 2), jnp.bfloat16),
            pltpu.VMEM((g, n), jnp.bfloat16),
        ],
        name="graphsage_fused",
    )(w1, b1, wl, bl, head_w1, head_b1, head_w2, head_b2, head_w3, head_b3,
      sc_x, sc_adj, fc_x, fc_adj, pool_mat)
    return out
